# baseline (speedup 1.0000x reference)
import functools

import jax
import jax.numpy as jnp
import numpy as np
from jax import lax
from jax.experimental import pallas as pl
from jax.experimental.pallas import tpu as pltpu
from jax.experimental.pallas import tpu_sc as plsc

N = 10000
E = 320000
D = 128
BN_EPS = 1e-5

NC = 2
NS = 16
NW = NC * NS
CHUNK = 128
NV = CHUNK // 16
NCH0 = 80
NCH1 = 80
TOT_CH = NS * (NCH0 + NCH1)
EP = TOT_CH * CHUNK
PK_PAD = (NCH0 - NCH1) * CHUNK
DST_SHIFT = 14
ACC_ROWS = 10112
ROWS_PER_TILE = ACC_ROWS // NS

_PACK_BLK = 4096
_PACK_GRID = EP // _PACK_BLK
_TAIL_BLKS = (EP - E) // _PACK_BLK + 1
_p = np.arange((_PACK_GRID - _TAIL_BLKS) * _PACK_BLK, EP, dtype=np.int64)
_pi = np.maximum(_p - E, 0).astype(np.int32)
_PK_TAIL = ((_pi % N) | ((N + _pi % (ACC_ROWS - N)) << DST_SHIFT)
            ).reshape(_TAIL_BLKS, 1, _PACK_BLK)

def _sc_body(x_hbm, pk_hbm, zeros_hbm, out_hbm,
             pk, usrc, udst, rows0, rows1, acc,
             gsem0, gsem1, ssem0, ssem1, zsem):
    cid = lax.axis_index("c")
    sid = lax.axis_index("s")
    bufs = (rows0, rows1)
    gsems = (gsem0, gsem1)
    ssems = (ssem0, ssem1)

    ncht = lax.select(cid == 0, NCH0, NCH1)
    off_ch = lax.select(cid == 0, NS * NCH1 + sid * NCH0, sid * NCH1)
    off = pl.multiple_of(off_ch * CHUNK, CHUNK)

    stripe = pl.ds(sid * ROWS_PER_TILE, ROWS_PER_TILE)
    zdesc = pltpu.async_copy(zeros_hbm, acc.at[stripe], zsem)
    pltpu.sync_copy(pk_hbm.at[pl.ds(off, NCH0 * CHUNK)], pk)
    zdesc.wait()

    plsc.subcore_barrier()

    def unpack(c, b, buf, shift, mask):
        base = pl.multiple_of(c * CHUNK, CHUNK)
        for j in range(NV):
            v = pk[pl.ds(base + j * 16, 16)]
            buf[b, pl.ds(j * 16, 16)] = (v >> shift) & mask

    def unpack_src(c, b):
        unpack(c, b, usrc, 0, (1 << DST_SHIFT) - 1)

    def unpack_dst(c, b):
        unpack(c, b, udst, DST_SHIFT, (1 << (30 - DST_SHIFT)) - 1)

    def fire_gather(c, s):
        unpack_src(c, s)
        pltpu.async_copy(x_hbm.at[usrc.at[s]], bufs[s], gsems[s])

    def drain_gather(s):
        pltpu.make_async_copy(
            x_hbm.at[usrc.at[s]], bufs[s], gsems[s]).wait()

    def fire_scatter(c, s):
        unpack_dst(c, s)
        pltpu.async_copy(
            bufs[s], acc.at[udst.at[s]], ssems[s], add=True)

    def drain_scatter(s):
        pltpu.make_async_copy(
            bufs[s], acc.at[udst.at[s]], ssems[s]).wait()

    def chunk(c, s, first=False, last=False):
        if not first:
            drain_scatter(1 - s)
        if not last:
            fire_gather(c + 1, 1 - s)
        drain_gather(s)
        fire_scatter(c, s)

    fire_gather(0, 0)
    chunk(0, 0, first=True)
    chunk(1, 1)

    def steady(p, carry):
        chunk(2 * p, 0)
        chunk(2 * p + 1, 1)
        return carry

    lax.fori_loop(1, ncht // 2 - 1, steady, 0)

    chunk(ncht - 2, 0)
    chunk(ncht - 1, 1, last=True)
    drain_scatter(1)

    plsc.subcore_barrier()

    pltpu.sync_copy(acc.at[stripe], out_hbm.at[cid].at[stripe])


@functools.cache
def _sc_segment_sum():
    mesh = plsc.VectorSubcoreMesh(
        core_axis_name="c", subcore_axis_name="s",
        num_cores=NC, num_subcores=NS)
    return pl.kernel(
        _sc_body,
        out_type=jax.ShapeDtypeStruct((NC, ACC_ROWS, D), jnp.float32),
        mesh=mesh,
        scratch_types=[
            pltpu.VMEM((NCH0 * CHUNK,), jnp.int32),
            pltpu.VMEM((2, CHUNK), jnp.int32),
            pltpu.VMEM((2, CHUNK), jnp.int32),
            pltpu.VMEM((CHUNK, D), jnp.float32),
            pltpu.VMEM((CHUNK, D), jnp.float32),
            pltpu.VMEM_SHARED((ACC_ROWS, D), jnp.float32),
            pltpu.SemaphoreType.DMA,
            pltpu.SemaphoreType.DMA,
            pltpu.SemaphoreType.DMA,
            pltpu.SemaphoreType.DMA,
            pltpu.SemaphoreType.DMA,
        ],
    )


_BLK = 1000
_GRID = N // _BLK


def _pack_body(e_ref, t_ref, out_ref):
    i = pl.program_id(0)
    s = e_ref[0:1, :].reshape(1, 1, _PACK_BLK)
    d = e_ref[1:2, :].reshape(1, 1, _PACK_BLK)
    pos = i * _PACK_BLK + lax.broadcasted_iota(
        jnp.int32, (1, 1, _PACK_BLK), 2)
    out_ref[...] = jnp.where(pos < E, s | (d << DST_SHIFT), t_ref[...])


def _tc_body(eps_ref, x_ref, n0_ref, n1_ref, w1_ref, b1_ref, w2_ref,
             b2_ref, g_ref, be_ref, out_ref, h2_scr, st_scr):
    i = pl.program_id(0)

    @pl.when(i < _GRID)
    def _():
        eps = eps_ref[0]
        m = (1.0 + eps) * x_ref[...] + n0_ref[0] + n1_ref[0]
        a1 = jnp.maximum(
            jnp.dot(m, w1_ref[...], preferred_element_type=jnp.float32)
            + b1_ref[...], 0.0)
        h2 = (jnp.dot(a1, w2_ref[...], preferred_element_type=jnp.float32)
              + b2_ref[...])
        h2_scr[pl.ds(i * _BLK, _BLK), :] = h2
        s1 = jnp.sum(h2, axis=0, keepdims=True)
        s2 = jnp.sum(h2 * h2, axis=0, keepdims=True)
        blk = jnp.concatenate(
            [s1, s2, jnp.zeros((6, D), jnp.float32)], axis=0)

        @pl.when(i == 0)
        def _():
            st_scr[...] = blk

        @pl.when(i > 0)
        def _():
            st_scr[...] += blk

    @pl.when(i >= _GRID)
    def _():
        j = i - _GRID
        h2 = h2_scr[pl.ds(j * _BLK, _BLK), :]
        mean = st_scr[0:1, :] / N
        var = st_scr[1:2, :] / N - mean * mean
        inv = lax.rsqrt(var + BN_EPS)
        h = g_ref[...] * (h2 - mean) * inv + be_ref[...]
        out_ref[...] = x_ref[...] + jnp.maximum(h, 0.0)


def kernel(x, edge_index, W1, b1, W2, b2, gamma, beta, eps):
    packed = pl.pallas_call(
        _pack_body,
        grid=(_PACK_GRID,),
        in_specs=[
            pl.BlockSpec(
                (2, _PACK_BLK),
                lambda i: (0, jnp.minimum(i, (E - 1) // _PACK_BLK))),
            pl.BlockSpec(
                (1, 1, _PACK_BLK),
                lambda i: (jnp.maximum(i - (_PACK_GRID - _TAIL_BLKS), 0),
                           0, 0)),
        ],
        out_specs=pl.BlockSpec((1, 1, _PACK_BLK), lambda i: (i, 0, 0)),
        out_shape=jax.ShapeDtypeStruct(
            (_PACK_GRID, 1, _PACK_BLK), jnp.int32),
    )(edge_index, jnp.asarray(_PK_TAIL)).reshape(EP)
    zeros = jnp.zeros((ROWS_PER_TILE, D), jnp.float32)

    nacc = _sc_segment_sum()(x, packed, zeros)

    def _row_ix(i):
        return (jnp.where(i < _GRID, i, i - _GRID), 0)

    row_spec = pl.BlockSpec((_BLK, D), _row_ix)
    nacc0_spec = pl.BlockSpec(
        (1, _BLK, D), lambda i: (0, jnp.where(i < _GRID, i, 0), 0))
    nacc1_spec = pl.BlockSpec(
        (1, _BLK, D), lambda i: (1, jnp.where(i < _GRID, i, 0), 0))
    full_mat = pl.BlockSpec((D, D), lambda i: (0, 0))
    full_vec = pl.BlockSpec((1, D), lambda i: (0, 0))

    out = pl.pallas_call(
        _tc_body,
        grid=(2 * _GRID,),
        in_specs=[
            pl.BlockSpec(memory_space=pltpu.SMEM),
            row_spec, nacc0_spec, nacc1_spec,
            full_mat, full_vec, full_mat, full_vec,
            full_vec, full_vec,
        ],
        out_specs=row_spec,
        out_shape=jax.ShapeDtypeStruct((N, D), jnp.float32),
        scratch_shapes=[
            pltpu.VMEM((N, D), jnp.float32),
            pltpu.VMEM((8, D), jnp.float32),
        ],
    )(eps.reshape(1), x, nacc, nacc, W1, b1.reshape(1, D),
      W2, b2.reshape(1, D), gamma.reshape(1, D), beta.reshape(1, D))

    return out


# speedup vs baseline: 1.2109x; 1.2109x over previous
import functools

import jax
import jax.numpy as jnp
import numpy as np
from jax import lax
from jax.experimental import pallas as pl
from jax.experimental.pallas import tpu as pltpu
from jax.experimental.pallas import tpu_sc as plsc

N = 10000
E = 320000
D = 128
BN_EPS = 1e-5

NC = 2
NS = 16
NW = NC * NS
CHUNK = 128
NV = CHUNK // 16
NCH0 = 80
NCH1 = 80
TOT_CH = NS * (NCH0 + NCH1)
EP = TOT_CH * CHUNK
PK_PAD = (NCH0 - NCH1) * CHUNK
DST_SHIFT = 14
ACC_ROWS = 10112
ROWS_PER_TILE = ACC_ROWS // NS

_PACK_BLK = 32768
_PACK_GRID = EP // _PACK_BLK
_TAIL_BLKS = (EP - E) // _PACK_BLK + 1
_p = np.arange((_PACK_GRID - _TAIL_BLKS) * _PACK_BLK, EP, dtype=np.int64)
_pi = np.maximum(_p - E, 0).astype(np.int32)
_PK_TAIL = ((_pi % N) | ((N + _pi % (ACC_ROWS - N)) << DST_SHIFT)
            ).reshape(_TAIL_BLKS, 1, _PACK_BLK)

def _sc_body(x_hbm, pk_hbm, zeros_hbm, out_hbm,
             pk, usrc, udst, rows0, rows1, acc,
             gsem0, gsem1, ssem0, ssem1, zsem):
    cid = lax.axis_index("c")
    sid = lax.axis_index("s")
    bufs = (rows0, rows1)
    gsems = (gsem0, gsem1)
    ssems = (ssem0, ssem1)

    ncht = lax.select(cid == 0, NCH0, NCH1)
    off_ch = lax.select(cid == 0, NS * NCH1 + sid * NCH0, sid * NCH1)
    off = pl.multiple_of(off_ch * CHUNK, CHUNK)

    stripe = pl.ds(sid * ROWS_PER_TILE, ROWS_PER_TILE)
    zdesc = pltpu.async_copy(zeros_hbm, acc.at[stripe], zsem)
    pltpu.sync_copy(pk_hbm.at[pl.ds(off, NCH0 * CHUNK)], pk)
    zdesc.wait()

    plsc.subcore_barrier()

    def unpack(c, b, buf, shift, mask):
        base = pl.multiple_of(c * CHUNK, CHUNK)
        for j in range(NV):
            v = pk[pl.ds(base + j * 16, 16)]
            buf[b, pl.ds(j * 16, 16)] = (v >> shift) & mask

    def unpack_src(c, b):
        unpack(c, b, usrc, 0, (1 << DST_SHIFT) - 1)

    def unpack_dst(c, b):
        unpack(c, b, udst, DST_SHIFT, (1 << (30 - DST_SHIFT)) - 1)

    def fire_gather(c, s):
        unpack_src(c, s)
        pltpu.async_copy(x_hbm.at[usrc.at[s]], bufs[s], gsems[s])

    def drain_gather(s):
        pltpu.make_async_copy(
            x_hbm.at[usrc.at[s]], bufs[s], gsems[s]).wait()

    def fire_scatter(c, s):
        unpack_dst(c, s)
        pltpu.async_copy(
            bufs[s], acc.at[udst.at[s]], ssems[s], add=True)

    def drain_scatter(s):
        pltpu.make_async_copy(
            bufs[s], acc.at[udst.at[s]], ssems[s]).wait()

    def chunk(c, s, first=False, last=False):
        if not first:
            drain_scatter(1 - s)
        if not last:
            fire_gather(c + 1, 1 - s)
        drain_gather(s)
        fire_scatter(c, s)

    fire_gather(0, 0)
    chunk(0, 0, first=True)
    chunk(1, 1)

    def steady(p, carry):
        chunk(2 * p, 0)
        chunk(2 * p + 1, 1)
        return carry

    lax.fori_loop(1, ncht // 2 - 1, steady, 0)

    chunk(ncht - 2, 0)
    chunk(ncht - 1, 1, last=True)
    drain_scatter(1)

    plsc.subcore_barrier()

    pltpu.sync_copy(acc.at[stripe], out_hbm.at[cid].at[stripe])


@functools.cache
def _sc_segment_sum():
    mesh = plsc.VectorSubcoreMesh(
        core_axis_name="c", subcore_axis_name="s",
        num_cores=NC, num_subcores=NS)
    return pl.kernel(
        _sc_body,
        out_type=jax.ShapeDtypeStruct((NC, ACC_ROWS, D), jnp.float32),
        mesh=mesh,
        scratch_types=[
            pltpu.VMEM((NCH0 * CHUNK,), jnp.int32),
            pltpu.VMEM((2, CHUNK), jnp.int32),
            pltpu.VMEM((2, CHUNK), jnp.int32),
            pltpu.VMEM((CHUNK, D), jnp.float32),
            pltpu.VMEM((CHUNK, D), jnp.float32),
            pltpu.VMEM_SHARED((ACC_ROWS, D), jnp.float32),
            pltpu.SemaphoreType.DMA,
            pltpu.SemaphoreType.DMA,
            pltpu.SemaphoreType.DMA,
            pltpu.SemaphoreType.DMA,
            pltpu.SemaphoreType.DMA,
        ],
    )


_BLK = 1000
_GRID = N // _BLK


def _pack_body(e_ref, t_ref, out_ref):
    i = pl.program_id(0)
    s = e_ref[0:1, :].reshape(1, 1, _PACK_BLK)
    d = e_ref[1:2, :].reshape(1, 1, _PACK_BLK)
    pos = i * _PACK_BLK + lax.broadcasted_iota(
        jnp.int32, (1, 1, _PACK_BLK), 2)
    out_ref[...] = jnp.where(pos < E, s | (d << DST_SHIFT), t_ref[...])


def _tc_body(eps_ref, x_ref, n0_ref, n1_ref, w1_ref, b1_ref, w2_ref,
             b2_ref, g_ref, be_ref, out_ref, h2_scr, st_scr):
    i = pl.program_id(0)

    @pl.when(i < _GRID)
    def _():
        eps = eps_ref[0]
        m = (1.0 + eps) * x_ref[...] + n0_ref[0] + n1_ref[0]
        a1 = jnp.maximum(
            jnp.dot(m, w1_ref[...], preferred_element_type=jnp.float32)
            + b1_ref[...], 0.0)
        h2 = (jnp.dot(a1, w2_ref[...], preferred_element_type=jnp.float32)
              + b2_ref[...])
        h2_scr[pl.ds(i * _BLK, _BLK), :] = h2
        s1 = jnp.sum(h2, axis=0, keepdims=True)
        s2 = jnp.sum(h2 * h2, axis=0, keepdims=True)
        blk = jnp.concatenate(
            [s1, s2, jnp.zeros((6, D), jnp.float32)], axis=0)

        @pl.when(i == 0)
        def _():
            st_scr[...] = blk

        @pl.when(i > 0)
        def _():
            st_scr[...] += blk

    @pl.when(i >= _GRID)
    def _():
        j = i - _GRID
        h2 = h2_scr[pl.ds(j * _BLK, _BLK), :]
        mean = st_scr[0:1, :] / N
        var = st_scr[1:2, :] / N - mean * mean
        inv = lax.rsqrt(var + BN_EPS)
        h = g_ref[...] * (h2 - mean) * inv + be_ref[...]
        out_ref[...] = x_ref[...] + jnp.maximum(h, 0.0)


def kernel(x, edge_index, W1, b1, W2, b2, gamma, beta, eps):
    packed = pl.pallas_call(
        _pack_body,
        grid=(_PACK_GRID,),
        in_specs=[
            pl.BlockSpec(
                (2, _PACK_BLK),
                lambda i: (0, jnp.minimum(i, (E - 1) // _PACK_BLK))),
            pl.BlockSpec(
                (1, 1, _PACK_BLK),
                lambda i: (jnp.maximum(i - (_PACK_GRID - _TAIL_BLKS), 0),
                           0, 0)),
        ],
        out_specs=pl.BlockSpec((1, 1, _PACK_BLK), lambda i: (i, 0, 0)),
        out_shape=jax.ShapeDtypeStruct(
            (_PACK_GRID, 1, _PACK_BLK), jnp.int32),
    )(edge_index, jnp.asarray(_PK_TAIL)).reshape(EP)
    zeros = jnp.zeros((ROWS_PER_TILE, D), jnp.float32)

    nacc = _sc_segment_sum()(x, packed, zeros)

    def _row_ix(i):
        return (jnp.where(i < _GRID, i, i - _GRID), 0)

    row_spec = pl.BlockSpec((_BLK, D), _row_ix)
    nacc0_spec = pl.BlockSpec(
        (1, _BLK, D), lambda i: (0, jnp.where(i < _GRID, i, 0), 0))
    nacc1_spec = pl.BlockSpec(
        (1, _BLK, D), lambda i: (1, jnp.where(i < _GRID, i, 0), 0))
    full_mat = pl.BlockSpec((D, D), lambda i: (0, 0))
    full_vec = pl.BlockSpec((1, D), lambda i: (0, 0))

    out = pl.pallas_call(
        _tc_body,
        grid=(2 * _GRID,),
        in_specs=[
            pl.BlockSpec(memory_space=pltpu.SMEM),
            row_spec, nacc0_spec, nacc1_spec,
            full_mat, full_vec, full_mat, full_vec,
            full_vec, full_vec,
        ],
        out_specs=row_spec,
        out_shape=jax.ShapeDtypeStruct((N, D), jnp.float32),
        scratch_shapes=[
            pltpu.VMEM((N, D), jnp.float32),
            pltpu.VMEM((8, D), jnp.float32),
        ],
    )(eps.reshape(1), x, nacc, nacc, W1, b1.reshape(1, D),
      W2, b2.reshape(1, D), gamma.reshape(1, D), beta.reshape(1, D))

    return out


# speedup vs baseline: 1.2273x; 1.0135x over previous
import functools

import jax
import jax.numpy as jnp
import numpy as np
from jax import lax
from jax.experimental import pallas as pl
from jax.experimental.pallas import tpu as pltpu
from jax.experimental.pallas import tpu_sc as plsc

N = 10000
E = 320000
D = 128
BN_EPS = 1e-5

NC = 2
NS = 16
NW = NC * NS
CHUNK = 128
NV = CHUNK // 16
NCH0 = 80
NCH1 = 80
TOT_CH = NS * (NCH0 + NCH1)
EP = TOT_CH * CHUNK
PK_PAD = (NCH0 - NCH1) * CHUNK
DST_SHIFT = 14
ACC_ROWS = 10112
ROWS_PER_TILE = ACC_ROWS // NS

_PACK_BLK = 65536
_PACK_GRID = EP // _PACK_BLK
_TAIL_BLKS = (EP - E) // _PACK_BLK + 1
_p = np.arange((_PACK_GRID - _TAIL_BLKS) * _PACK_BLK, EP, dtype=np.int64)
_pi = np.maximum(_p - E, 0).astype(np.int32)
_PK_TAIL = ((_pi % N) | ((N + _pi % (ACC_ROWS - N)) << DST_SHIFT)
            ).reshape(_TAIL_BLKS, 1, _PACK_BLK)

def _sc_body(x_hbm, pk_hbm, zeros_hbm, out_hbm,
             pk, usrc, udst, rows0, rows1, acc,
             gsem0, gsem1, ssem0, ssem1, zsem):
    cid = lax.axis_index("c")
    sid = lax.axis_index("s")
    bufs = (rows0, rows1)
    gsems = (gsem0, gsem1)
    ssems = (ssem0, ssem1)

    ncht = lax.select(cid == 0, NCH0, NCH1)
    off_ch = lax.select(cid == 0, NS * NCH1 + sid * NCH0, sid * NCH1)
    off = pl.multiple_of(off_ch * CHUNK, CHUNK)

    stripe = pl.ds(sid * ROWS_PER_TILE, ROWS_PER_TILE)
    zdesc = pltpu.async_copy(zeros_hbm, acc.at[stripe], zsem)
    pltpu.sync_copy(pk_hbm.at[pl.ds(off, NCH0 * CHUNK)], pk)
    zdesc.wait()

    plsc.subcore_barrier()

    def unpack(c, b, buf, shift, mask):
        base = pl.multiple_of(c * CHUNK, CHUNK)
        for j in range(NV):
            v = pk[pl.ds(base + j * 16, 16)]
            buf[b, pl.ds(j * 16, 16)] = (v >> shift) & mask

    def unpack_src(c, b):
        unpack(c, b, usrc, 0, (1 << DST_SHIFT) - 1)

    def unpack_dst(c, b):
        unpack(c, b, udst, DST_SHIFT, (1 << (30 - DST_SHIFT)) - 1)

    def fire_gather(c, s):
        unpack_src(c, s)
        pltpu.async_copy(x_hbm.at[usrc.at[s]], bufs[s], gsems[s])

    def drain_gather(s):
        pltpu.make_async_copy(
            x_hbm.at[usrc.at[s]], bufs[s], gsems[s]).wait()

    def fire_scatter(c, s):
        unpack_dst(c, s)
        pltpu.async_copy(
            bufs[s], acc.at[udst.at[s]], ssems[s], add=True)

    def drain_scatter(s):
        pltpu.make_async_copy(
            bufs[s], acc.at[udst.at[s]], ssems[s]).wait()

    def chunk(c, s, first=False, last=False):
        if not first:
            drain_scatter(1 - s)
        if not last:
            fire_gather(c + 1, 1 - s)
        drain_gather(s)
        fire_scatter(c, s)

    fire_gather(0, 0)
    chunk(0, 0, first=True)
    chunk(1, 1)

    def steady(p, carry):
        chunk(2 * p, 0)
        chunk(2 * p + 1, 1)
        return carry

    lax.fori_loop(1, ncht // 2 - 1, steady, 0)

    chunk(ncht - 2, 0)
    chunk(ncht - 1, 1, last=True)
    drain_scatter(1)

    plsc.subcore_barrier()

    pltpu.sync_copy(acc.at[stripe], out_hbm.at[cid].at[stripe])


@functools.cache
def _sc_segment_sum():
    mesh = plsc.VectorSubcoreMesh(
        core_axis_name="c", subcore_axis_name="s",
        num_cores=NC, num_subcores=NS)
    return pl.kernel(
        _sc_body,
        out_type=jax.ShapeDtypeStruct((NC, ACC_ROWS, D), jnp.float32),
        mesh=mesh,
        scratch_types=[
            pltpu.VMEM((NCH0 * CHUNK,), jnp.int32),
            pltpu.VMEM((2, CHUNK), jnp.int32),
            pltpu.VMEM((2, CHUNK), jnp.int32),
            pltpu.VMEM((CHUNK, D), jnp.float32),
            pltpu.VMEM((CHUNK, D), jnp.float32),
            pltpu.VMEM_SHARED((ACC_ROWS, D), jnp.float32),
            pltpu.SemaphoreType.DMA,
            pltpu.SemaphoreType.DMA,
            pltpu.SemaphoreType.DMA,
            pltpu.SemaphoreType.DMA,
            pltpu.SemaphoreType.DMA,
        ],
    )


_BLK = 1000
_GRID = N // _BLK


def _pack_body(e_ref, t_ref, out_ref):
    i = pl.program_id(0)
    s = e_ref[0:1, :].reshape(1, 1, _PACK_BLK)
    d = e_ref[1:2, :].reshape(1, 1, _PACK_BLK)
    pos = i * _PACK_BLK + lax.broadcasted_iota(
        jnp.int32, (1, 1, _PACK_BLK), 2)
    out_ref[...] = jnp.where(pos < E, s | (d << DST_SHIFT), t_ref[...])


def _tc_body(eps_ref, x_ref, n0_ref, n1_ref, w1_ref, b1_ref, w2_ref,
             b2_ref, g_ref, be_ref, out_ref, h2_scr, st_scr):
    i = pl.program_id(0)

    @pl.when(i < _GRID)
    def _():
        eps = eps_ref[0]
        m = (1.0 + eps) * x_ref[...] + n0_ref[0] + n1_ref[0]
        a1 = jnp.maximum(
            jnp.dot(m, w1_ref[...], preferred_element_type=jnp.float32)
            + b1_ref[...], 0.0)
        h2 = (jnp.dot(a1, w2_ref[...], preferred_element_type=jnp.float32)
              + b2_ref[...])
        h2_scr[pl.ds(i * _BLK, _BLK), :] = h2
        s1 = jnp.sum(h2, axis=0, keepdims=True)
        s2 = jnp.sum(h2 * h2, axis=0, keepdims=True)
        blk = jnp.concatenate(
            [s1, s2, jnp.zeros((6, D), jnp.float32)], axis=0)

        @pl.when(i == 0)
        def _():
            st_scr[...] = blk

        @pl.when(i > 0)
        def _():
            st_scr[...] += blk

    @pl.when(i >= _GRID)
    def _():
        j = i - _GRID
        h2 = h2_scr[pl.ds(j * _BLK, _BLK), :]
        mean = st_scr[0:1, :] / N
        var = st_scr[1:2, :] / N - mean * mean
        inv = lax.rsqrt(var + BN_EPS)
        h = g_ref[...] * (h2 - mean) * inv + be_ref[...]
        out_ref[...] = x_ref[...] + jnp.maximum(h, 0.0)


def kernel(x, edge_index, W1, b1, W2, b2, gamma, beta, eps):
    packed = pl.pallas_call(
        _pack_body,
        grid=(_PACK_GRID,),
        in_specs=[
            pl.BlockSpec(
                (2, _PACK_BLK),
                lambda i: (0, jnp.minimum(i, (E - 1) // _PACK_BLK))),
            pl.BlockSpec(
                (1, 1, _PACK_BLK),
                lambda i: (jnp.maximum(i - (_PACK_GRID - _TAIL_BLKS), 0),
                           0, 0)),
        ],
        out_specs=pl.BlockSpec((1, 1, _PACK_BLK), lambda i: (i, 0, 0)),
        out_shape=jax.ShapeDtypeStruct(
            (_PACK_GRID, 1, _PACK_BLK), jnp.int32),
    )(edge_index, jnp.asarray(_PK_TAIL)).reshape(EP)
    zeros = jnp.zeros((ROWS_PER_TILE, D), jnp.float32)

    nacc = _sc_segment_sum()(x, packed, zeros)

    def _row_ix(i):
        return (jnp.where(i < _GRID, i, i - _GRID), 0)

    row_spec = pl.BlockSpec((_BLK, D), _row_ix)
    nacc0_spec = pl.BlockSpec(
        (1, _BLK, D), lambda i: (0, jnp.where(i < _GRID, i, 0), 0))
    nacc1_spec = pl.BlockSpec(
        (1, _BLK, D), lambda i: (1, jnp.where(i < _GRID, i, 0), 0))
    full_mat = pl.BlockSpec((D, D), lambda i: (0, 0))
    full_vec = pl.BlockSpec((1, D), lambda i: (0, 0))

    out = pl.pallas_call(
        _tc_body,
        grid=(2 * _GRID,),
        in_specs=[
            pl.BlockSpec(memory_space=pltpu.SMEM),
            row_spec, nacc0_spec, nacc1_spec,
            full_mat, full_vec, full_mat, full_vec,
            full_vec, full_vec,
        ],
        out_specs=row_spec,
        out_shape=jax.ShapeDtypeStruct((N, D), jnp.float32),
        scratch_shapes=[
            pltpu.VMEM((N, D), jnp.float32),
            pltpu.VMEM((8, D), jnp.float32),
        ],
    )(eps.reshape(1), x, nacc, nacc, W1, b1.reshape(1, D),
      W2, b2.reshape(1, D), gamma.reshape(1, D), beta.reshape(1, D))

    return out
